# Initial kernel scaffold; baseline (speedup 1.0000x reference)
import jax
import jax.numpy as jnp
from jax.experimental import pallas as pl


def kernel(x, hidden, emb, Wx_f, Wh_f, b_f, Wx_b, Wh_b, b_b):
    raise NotImplementedError("write your pallas kernel here")


# speedup vs baseline: 2.3038x; 2.3038x over previous
import jax
import jax.numpy as jnp
from jax.experimental import pallas as pl
from jax.experimental.pallas import tpu as pltpu
from jax.experimental.pallas import tpu_sc as plsc

V = 100000
D = 128
U = 256
B = 1024
T = 50
H4 = 4 * U
_GW = 128


def _sc_gather(emb, idx):
    n = idx.shape[0]
    mesh = plsc.VectorSubcoreMesh(core_axis_name="core", subcore_axis_name="subcore")

    @pl.kernel(out_type=jax.ShapeDtypeStruct((n, D), emb.dtype), mesh=mesh)
    def gather_kernel(x_hbm, i_hbm, o_hbm):
        def body(i_vmem, o_vmem):
            pltpu.sync_copy(x_hbm.at[i_vmem.at[0]], o_vmem)

        pltpu.emit_pipeline(
            body,
            grid=(n // _GW,),
            in_specs=[pl.BlockSpec((1, _GW), index_map=lambda i: (0, i))],
            out_specs=[pl.BlockSpec((_GW, D), index_map=lambda i: (i, 0))],
            core_axis_name=("core", "subcore"),
            dimension_semantics=(pltpu.PARALLEL,),
        )(i_hbm, o_hbm)

    return gather_kernel(emb, idx.reshape(1, n))


def _lstm_body(e_ref, h0_ref, wx_ref, wh_ref, b_ref, ys_ref, st_ref, h_sc, c_sc):
    t = pl.program_id(1)

    @pl.when(t == 0)
    def _():
        h_sc[...] = h0_ref[...]
        c_sc[...] = jnp.zeros_like(c_sc)

    z = jnp.dot(e_ref[0], wx_ref[0], preferred_element_type=jnp.float32)
    z = z + jnp.dot(h_sc[...], wh_ref[0], preferred_element_type=jnp.float32)
    z = z + b_ref[0]
    i = jax.nn.sigmoid(z[:, :U])
    f = jax.nn.sigmoid(z[:, U:2 * U])
    g = jnp.tanh(z[:, 2 * U:3 * U])
    o = jax.nn.sigmoid(z[:, 3 * U:])
    c = f * c_sc[...] + i * g
    h = o * jnp.tanh(c)
    c_sc[...] = c
    h_sc[...] = h
    ys_ref[...] = h

    @pl.when(t == T - 1)
    def _():
        st_ref[...] = h


def _lstm_tc(e_tm, hidden, wx_s, wh_s, b_s):
    return pl.pallas_call(
        _lstm_body,
        grid=(2, T),
        in_specs=[
            pl.BlockSpec((1, B, D), lambda d, t: (jnp.where(d == 0, t, T - 1 - t), 0, 0)),
            pl.BlockSpec((B, U), lambda d, t: (0, 0)),
            pl.BlockSpec((1, D, H4), lambda d, t: (d, 0, 0)),
            pl.BlockSpec((1, U, H4), lambda d, t: (d, 0, 0)),
            pl.BlockSpec((1, 1, H4), lambda d, t: (d, 0, 0)),
        ],
        out_specs=[
            pl.BlockSpec(
                (B, U),
                lambda d, t: (0, jnp.where(d == 0, 2 * t, 2 * (T - 1 - t) + 1)),
            ),
            pl.BlockSpec((B, U), lambda d, t: (0, d)),
        ],
        out_shape=[
            jax.ShapeDtypeStruct((B, T * 2 * U), jnp.float32),
            jax.ShapeDtypeStruct((B, 2 * U), jnp.float32),
        ],
        scratch_shapes=[
            pltpu.VMEM((B, U), jnp.float32),
            pltpu.VMEM((B, U), jnp.float32),
        ],
        compiler_params=pltpu.CompilerParams(
            dimension_semantics=("arbitrary", "arbitrary"),
        ),
    )(e_tm, hidden, wx_s, wh_s, b_s)


def kernel(x, hidden, emb, Wx_f, Wh_f, b_f, Wx_b, Wh_b, b_b):
    idx_tm = x.astype(jnp.int32).T.reshape(-1)
    e_tm = _sc_gather(emb, idx_tm).reshape(T, B, D)
    wx_s = jnp.stack([Wx_f, Wx_b])
    wh_s = jnp.stack([Wh_f, Wh_b])
    b_s = jnp.stack([b_f, b_b]).reshape(2, 1, H4)
    ys, state = _lstm_tc(e_tm, hidden, wx_s, wh_s, b_s)
    return (ys.reshape(B, T, 2 * U), state)


# speedup vs baseline: 2.4428x; 1.0603x over previous
import jax
import jax.numpy as jnp
from jax.experimental import pallas as pl
from jax.experimental.pallas import tpu as pltpu
from jax.experimental.pallas import tpu_sc as plsc

V = 100000
D = 128
U = 256
B = 1024
T = 50
H4 = 4 * U
_GW = 128


def _sc_gather(emb, idx):
    n = idx.shape[0]
    mesh = plsc.VectorSubcoreMesh(core_axis_name="core", subcore_axis_name="subcore")

    @pl.kernel(out_type=jax.ShapeDtypeStruct((n, D), emb.dtype), mesh=mesh)
    def gather_kernel(x_hbm, i_hbm, o_hbm):
        def body(i_vmem, o_vmem):
            pltpu.sync_copy(x_hbm.at[i_vmem.at[0]], o_vmem)

        pltpu.emit_pipeline(
            body,
            grid=(n // _GW,),
            in_specs=[pl.BlockSpec((1, _GW), index_map=lambda i: (0, i))],
            out_specs=[pl.BlockSpec((_GW, D), index_map=lambda i: (i, 0))],
            core_axis_name=("core", "subcore"),
            dimension_semantics=(pltpu.PARALLEL,),
        )(i_hbm, o_hbm)

    return gather_kernel(emb, idx.reshape(1, n))


def _lstm_body(e_ref, h0_ref, wx_ref, wh_ref, b_ref, ys_ref, st_ref, h_sc, c_sc):
    t = pl.program_id(1)

    @pl.when(t == 0)
    def _():
        h_sc[...] = h0_ref[...]
        c_sc[...] = jnp.zeros_like(c_sc)

    z = jnp.dot(e_ref[0], wx_ref[0], preferred_element_type=jnp.float32,
                precision=jax.lax.Precision.DEFAULT)
    z = z + jnp.dot(h_sc[...], wh_ref[0], preferred_element_type=jnp.float32,
                    precision=jax.lax.Precision.DEFAULT)
    z = z + b_ref[0]
    i = 0.5 * jnp.tanh(0.5 * z[:, :U]) + 0.5
    f = 0.5 * jnp.tanh(0.5 * z[:, U:2 * U]) + 0.5
    g = jnp.tanh(z[:, 2 * U:3 * U])
    o = 0.5 * jnp.tanh(0.5 * z[:, 3 * U:]) + 0.5
    c = f * c_sc[...] + i * g
    h = o * jnp.tanh(c)
    c_sc[...] = c
    h_sc[...] = h
    ys_ref[...] = h

    @pl.when(t == T - 1)
    def _():
        st_ref[...] = h


def _lstm_tc(e_tm, hidden, wx_s, wh_s, b_s):
    return pl.pallas_call(
        _lstm_body,
        grid=(2, T),
        in_specs=[
            pl.BlockSpec((1, B, D), lambda d, t: (jnp.where(d == 0, t, T - 1 - t), 0, 0)),
            pl.BlockSpec((B, U), lambda d, t: (0, 0)),
            pl.BlockSpec((1, D, H4), lambda d, t: (d, 0, 0)),
            pl.BlockSpec((1, U, H4), lambda d, t: (d, 0, 0)),
            pl.BlockSpec((1, 1, H4), lambda d, t: (d, 0, 0)),
        ],
        out_specs=[
            pl.BlockSpec(
                (B, U),
                lambda d, t: (0, jnp.where(d == 0, 2 * t, 2 * (T - 1 - t) + 1)),
            ),
            pl.BlockSpec((B, U), lambda d, t: (0, d)),
        ],
        out_shape=[
            jax.ShapeDtypeStruct((B, T * 2 * U), jnp.float32),
            jax.ShapeDtypeStruct((B, 2 * U), jnp.float32),
        ],
        scratch_shapes=[
            pltpu.VMEM((B, U), jnp.float32),
            pltpu.VMEM((B, U), jnp.float32),
        ],
        compiler_params=pltpu.CompilerParams(
            dimension_semantics=("arbitrary", "arbitrary"),
        ),
    )(e_tm, hidden, wx_s, wh_s, b_s)


def kernel(x, hidden, emb, Wx_f, Wh_f, b_f, Wx_b, Wh_b, b_b):
    idx_tm = x.astype(jnp.int32).T.reshape(-1)
    e_tm = _sc_gather(emb, idx_tm).reshape(T, B, D)
    wx_s = jnp.stack([Wx_f, Wx_b])
    wh_s = jnp.stack([Wh_f, Wh_b])
    b_s = jnp.stack([b_f, b_b]).reshape(2, 1, H4)
    ys, state = _lstm_tc(e_tm, hidden, wx_s, wh_s, b_s)
    return (ys.reshape(B, T, 2 * U), state)
